# Initial kernel scaffold; baseline (speedup 1.0000x reference)
import jax
import jax.numpy as jnp
from jax.experimental import pallas as pl


def kernel(x, edge_index, batch, W0, b0, W1, b1, Wf, bf):
    raise NotImplementedError("write your pallas kernel here")


# speedup vs baseline: 13.8615x; 13.8615x over previous
import jax
import jax.numpy as jnp
from jax import lax
from jax.experimental import pallas as pl
from jax.experimental.pallas import tpu as pltpu
from jax.experimental.pallas import tpu_sc as plsc

N = 10000
E = 320000
C = 128
NPAD = 10240
RPT = 640
K = 80
R = 1000
EPW = E // 32

_mesh = plsc.VectorSubcoreMesh(core_axis_name="c", subcore_axis_name="s")
f32 = jnp.float32


def _fill_vec(ref, n, val):
    def body(j, _):
        ref[pl.ds(j * 16, 16)] = jnp.full((16,), val, f32)
        return 0
    lax.fori_loop(0, n // 16, body, 0)


def _deg_body(dst_hbm, d_hbm, deg_s, ones_v, dst_v, wbuf):
    cid = lax.axis_index("c")
    sid = lax.axis_index("s")
    _fill_vec(ones_v, K, 1.0)
    _fill_vec(wbuf, RPT, 0.0)
    pltpu.sync_copy(wbuf, deg_s.at[pl.ds(sid * RPT, RPT)])
    plsc.subcore_barrier()

    wid = sid * 2 + cid

    def ebody(i, _):
        pltpu.sync_copy(dst_hbm.at[pl.ds(wid * EPW + i * K, K)], dst_v)
        pltpu.sync_copy(ones_v, deg_s.at[dst_v], add=True)
        return 0
    lax.fori_loop(0, EPW // K, ebody, 0)
    plsc.subcore_barrier()

    pltpu.sync_copy(deg_s.at[pl.ds(sid * RPT, RPT)], wbuf)
    pltpu.sync_copy(wbuf, d_hbm.at[cid, pl.ds(sid * RPT, RPT)])


_deg_call = pl.kernel(
    _deg_body,
    out_type=jax.ShapeDtypeStruct((2, NPAD), f32),
    mesh=_mesh,
    compiler_params=pltpu.CompilerParams(use_tc_tiling_on_sc=False, needs_layout_passes=False),
    scratch_types=[
        pltpu.VMEM_SHARED((NPAD,), f32),
        pltpu.VMEM((K,), f32),
        pltpu.VMEM((K,), jnp.int32),
        pltpu.VMEM((RPT,), f32),
    ],
)


def _edge_body(g_hbm, src_hbm, dst_hbm, p_hbm, acc_s, buf, src_v, dst_v, sem):
    cid = lax.axis_index("c")
    sid = lax.axis_index("s")

    def zrow(r, _):
        for c4 in range(C // 16):
            buf[r, pl.ds(c4 * 16, 16)] = jnp.zeros((16,), f32)
        return 0
    lax.fori_loop(0, K, zrow, 0)

    def zcp(k, _):
        pltpu.sync_copy(buf, acc_s.at[pl.ds(sid * RPT + k * K, K)])
        return 0
    lax.fori_loop(0, RPT // K, zcp, 0)
    plsc.subcore_barrier()

    wid = sid * 2 + cid

    def ebody(i, _):
        base = wid * EPW + i * K
        pltpu.sync_copy(src_hbm.at[pl.ds(base, K)], src_v)
        pltpu.sync_copy(dst_hbm.at[pl.ds(base, K)], dst_v)
        pltpu.async_copy(g_hbm.at[src_v], buf, sem).wait()
        pltpu.sync_copy(buf, acc_s.at[dst_v], add=True)
        return 0
    lax.fori_loop(0, EPW // K, ebody, 0)
    plsc.subcore_barrier()

    def wb(k, _):
        base = sid * RPT + k * K
        pltpu.sync_copy(acc_s.at[pl.ds(base, K)], buf)
        pltpu.sync_copy(buf, p_hbm.at[cid, pl.ds(base, K)])
        return 0
    lax.fori_loop(0, RPT // K, wb, 0)


_edge_call = pl.kernel(
    _edge_body,
    out_type=jax.ShapeDtypeStruct((2, NPAD, C), f32),
    mesh=_mesh,
    compiler_params=pltpu.CompilerParams(use_tc_tiling_on_sc=False, needs_layout_passes=False),
    scratch_types=[
        pltpu.VMEM_SHARED((NPAD, C), f32),
        pltpu.VMEM((K, C), f32),
        pltpu.VMEM((K,), jnp.int32),
        pltpu.VMEM((K,), jnp.int32),
        pltpu.SemaphoreType.DMA,
    ],
)


def _fin_body(gf_hbm, src_hbm, dst_hbm, a_hbm,
              acc_s, gf_v, vbuf, src_v, dst_v, wbuf):
    cid = lax.axis_index("c")
    sid = lax.axis_index("s")
    _fill_vec(wbuf, RPT, 0.0)
    pltpu.sync_copy(wbuf, acc_s.at[pl.ds(sid * RPT, RPT)])
    pltpu.sync_copy(gf_hbm, gf_v)
    plsc.subcore_barrier()

    wid = sid * 2 + cid

    def ebody(i, _):
        base = wid * EPW + i * K
        pltpu.sync_copy(src_hbm.at[pl.ds(base, K)], src_v)
        pltpu.sync_copy(dst_hbm.at[pl.ds(base, K)], dst_v)
        for j in range(K // 16):
            idx = src_v[pl.ds(j * 16, 16)]
            vbuf[pl.ds(j * 16, 16)] = plsc.load_gather(gf_v, [idx])
        pltpu.sync_copy(vbuf, acc_s.at[dst_v], add=True)
        return 0
    lax.fori_loop(0, EPW // K, ebody, 0)
    plsc.subcore_barrier()

    pltpu.sync_copy(acc_s.at[pl.ds(sid * RPT, RPT)], wbuf)
    pltpu.sync_copy(wbuf, a_hbm.at[cid, pl.ds(sid * RPT, RPT)])


_fin_call = pl.kernel(
    _fin_body,
    out_type=jax.ShapeDtypeStruct((2, NPAD), f32),
    mesh=_mesh,
    compiler_params=pltpu.CompilerParams(use_tc_tiling_on_sc=False, needs_layout_passes=False),
    scratch_types=[
        pltpu.VMEM_SHARED((NPAD,), f32),
        pltpu.VMEM((N,), f32),
        pltpu.VMEM((K,), f32),
        pltpu.VMEM((K,), jnp.int32),
        pltpu.VMEM((K,), jnp.int32),
        pltpu.VMEM((RPT,), f32),
    ],
)


def _tc1_body(x_ref, w_ref, d0_ref, d1_ref, g_ref, dinv_ref):
    dinv = lax.rsqrt(d0_ref[0] + d1_ref[0] + 1.0)
    g_ref[...] = jnp.dot(x_ref[...], w_ref[...],
                         preferred_element_type=f32) * dinv
    dinv_ref[...] = dinv


def _tc2_body(p0_ref, p1_ref, g_ref, dinv_ref, b_ref, w_ref, o_ref):
    dinv = dinv_ref[...]
    h = jnp.maximum(
        dinv * (p0_ref[0] + p1_ref[0] + g_ref[...]) + b_ref[...], 0.0)
    o_ref[...] = jnp.dot(h, w_ref[...], preferred_element_type=f32) * dinv


def _tc4_body(a0_ref, a1_ref, gf_ref, dinv_ref, bf_ref, out_ref):
    out_ref[...] = dinv_ref[...] * (a0_ref[0] + a1_ref[0] + gf_ref[...]) \
        + bf_ref[...]


def _row_spec(w):
    return pl.BlockSpec((R, w), lambda i: (i, 0))


def _const_spec(h, w):
    return pl.BlockSpec((h, w), lambda i: (0, 0))


def _half_spec(c, w):
    return pl.BlockSpec((1, R, w), lambda i, c=c: (c, i, 0))


_GRID = N // R

_tc1_call = pl.pallas_call(
    _tc1_body,
    grid=(_GRID,),
    in_specs=[_row_spec(C), _const_spec(C, C), _half_spec(0, 1),
              _half_spec(1, 1)],
    out_specs=[_row_spec(C), _row_spec(1)],
    out_shape=[jax.ShapeDtypeStruct((N, C), f32),
               jax.ShapeDtypeStruct((N, 1), f32)],
)


def _make_tc2(cout):
    return pl.pallas_call(
        _tc2_body,
        grid=(_GRID,),
        in_specs=[_half_spec(0, C), _half_spec(1, C), _row_spec(C),
                  _row_spec(1), _const_spec(1, C), _const_spec(C, cout)],
        out_specs=_row_spec(cout),
        out_shape=jax.ShapeDtypeStruct((N, cout), f32),
    )


_tc2_call = _make_tc2(C)
_tc3_call = _make_tc2(1)

_tc4_call = pl.pallas_call(
    _tc4_body,
    grid=(_GRID,),
    in_specs=[_half_spec(0, 1), _half_spec(1, 1), _row_spec(1), _row_spec(1),
              _const_spec(1, 1)],
    out_specs=_row_spec(1),
    out_shape=jax.ShapeDtypeStruct((N, 1), f32),
)


@jax.jit
def kernel(x, edge_index, batch, W0, b0, W1, b1, Wf, bf):
    src = edge_index[0]
    dst = edge_index[1]

    d = _deg_call(dst).reshape(2, NPAD, 1)
    g0, dinv = _tc1_call(x, W0, d, d)
    p = _edge_call(g0, src, dst)
    g1 = _tc2_call(p, p, g0, dinv, b0.reshape(1, C), W1)
    q = _edge_call(g1, src, dst)
    gf = _tc3_call(q, q, g1, dinv, b1.reshape(1, C), Wf)
    a = _fin_call(gf.reshape(N), src, dst).reshape(2, NPAD, 1)
    out = _tc4_call(a, a, gf, dinv, bf.reshape(1, 1))
    return out


# speedup vs baseline: 19.1080x; 1.3785x over previous
import jax
import jax.numpy as jnp
from jax import lax
from jax.experimental import pallas as pl
from jax.experimental.pallas import tpu as pltpu
from jax.experimental.pallas import tpu_sc as plsc

N = 10000
E = 320000
C = 128
NPAD = 10240
RPT = 640
K = 80
R = 1000
EPW = E // 32

_mesh = plsc.VectorSubcoreMesh(core_axis_name="c", subcore_axis_name="s")
f32 = jnp.float32


def _fill_vec(ref, n, val):
    def body(j, _):
        ref[pl.ds(j * 16, 16)] = jnp.full((16,), val, f32)
        return 0
    lax.fori_loop(0, n // 16, body, 0)


def _deg_body(dst_hbm, d_hbm, deg_s, ones_v, dst_v, wbuf):
    cid = lax.axis_index("c")
    sid = lax.axis_index("s")
    _fill_vec(ones_v, K, 1.0)
    _fill_vec(wbuf, RPT, 0.0)
    pltpu.sync_copy(wbuf, deg_s.at[pl.ds(sid * RPT, RPT)])
    plsc.subcore_barrier()

    wid = sid * 2 + cid

    def ebody(i, _):
        pltpu.sync_copy(dst_hbm.at[pl.ds(wid * EPW + i * K, K)], dst_v)
        pltpu.sync_copy(ones_v, deg_s.at[dst_v], add=True)
        return 0
    lax.fori_loop(0, EPW // K, ebody, 0)
    plsc.subcore_barrier()

    pltpu.sync_copy(deg_s.at[pl.ds(sid * RPT, RPT)], wbuf)
    pltpu.sync_copy(wbuf, d_hbm.at[cid, pl.ds(sid * RPT, RPT)])


_deg_call = pl.kernel(
    _deg_body,
    out_type=jax.ShapeDtypeStruct((2, NPAD), f32),
    mesh=_mesh,
    compiler_params=pltpu.CompilerParams(use_tc_tiling_on_sc=False, needs_layout_passes=False),
    scratch_types=[
        pltpu.VMEM_SHARED((NPAD,), f32),
        pltpu.VMEM((K,), f32),
        pltpu.VMEM((K,), jnp.int32),
        pltpu.VMEM((RPT,), f32),
    ],
)


def _edge_body(g_hbm, src_hbm, dst_hbm, p_hbm, acc_s,
               buf0, buf1, src_v0, src_v1, dst_v0, dst_v1, sem0, sem1):
    cid = lax.axis_index("c")
    sid = lax.axis_index("s")

    def zrow(r, _):
        for c4 in range(C // 16):
            buf0[r, pl.ds(c4 * 16, 16)] = jnp.zeros((16,), f32)
        return 0
    lax.fori_loop(0, K, zrow, 0)

    def zcp(k, _):
        pltpu.sync_copy(buf0, acc_s.at[pl.ds(sid * RPT + k * K, K)])
        return 0
    lax.fori_loop(0, RPT // K, zcp, 0)
    plsc.subcore_barrier()

    wid = sid * 2 + cid
    ebase = wid * EPW

    def fetch(block, sv, dv, bf, sem):
        pltpu.sync_copy(src_hbm.at[pl.ds(ebase + block * K, K)], sv)
        pltpu.sync_copy(dst_hbm.at[pl.ds(ebase + block * K, K)], dv)
        pltpu.async_copy(g_hbm.at[sv], bf, sem)

    def drain(sv, dv, bf, sem):
        pltpu.make_async_copy(g_hbm.at[sv], bf, sem).wait()
        pltpu.sync_copy(bf, acc_s.at[dv], add=True)

    fetch(0, src_v0, dst_v0, buf0, sem0)

    def pair(o, _):
        fetch(2 * o + 1, src_v1, dst_v1, buf1, sem1)
        drain(src_v0, dst_v0, buf0, sem0)
        fetch(2 * o + 2, src_v0, dst_v0, buf0, sem0)
        drain(src_v1, dst_v1, buf1, sem1)
        return 0
    lax.fori_loop(0, (EPW // K) // 2, pair, 0)
    drain(src_v0, dst_v0, buf0, sem0)
    plsc.subcore_barrier()

    def wb(k, _):
        base = sid * RPT + k * K
        pltpu.sync_copy(acc_s.at[pl.ds(base, K)], buf0)
        pltpu.sync_copy(buf0, p_hbm.at[cid, pl.ds(base, K)])
        return 0
    lax.fori_loop(0, RPT // K, wb, 0)


_edge_call = pl.kernel(
    _edge_body,
    out_type=jax.ShapeDtypeStruct((2, NPAD, C), f32),
    mesh=_mesh,
    compiler_params=pltpu.CompilerParams(use_tc_tiling_on_sc=False, needs_layout_passes=False),
    scratch_types=[
        pltpu.VMEM_SHARED((NPAD, C), f32),
        pltpu.VMEM((K, C), f32),
        pltpu.VMEM((K, C), f32),
        pltpu.VMEM((K,), jnp.int32),
        pltpu.VMEM((K,), jnp.int32),
        pltpu.VMEM((K,), jnp.int32),
        pltpu.VMEM((K,), jnp.int32),
        pltpu.SemaphoreType.DMA,
        pltpu.SemaphoreType.DMA,
    ],
)


def _fin_body(gf_hbm, src_hbm, dst_hbm, a_hbm,
              acc_s, gf_v, vbuf, src_v, dst_v, wbuf):
    cid = lax.axis_index("c")
    sid = lax.axis_index("s")
    _fill_vec(wbuf, RPT, 0.0)
    pltpu.sync_copy(wbuf, acc_s.at[pl.ds(sid * RPT, RPT)])
    pltpu.sync_copy(gf_hbm, gf_v)
    plsc.subcore_barrier()

    wid = sid * 2 + cid

    def ebody(i, _):
        base = wid * EPW + i * K
        pltpu.sync_copy(src_hbm.at[pl.ds(base, K)], src_v)
        pltpu.sync_copy(dst_hbm.at[pl.ds(base, K)], dst_v)
        for j in range(K // 16):
            idx = src_v[pl.ds(j * 16, 16)]
            vbuf[pl.ds(j * 16, 16)] = plsc.load_gather(gf_v, [idx])
        pltpu.sync_copy(vbuf, acc_s.at[dst_v], add=True)
        return 0
    lax.fori_loop(0, EPW // K, ebody, 0)
    plsc.subcore_barrier()

    pltpu.sync_copy(acc_s.at[pl.ds(sid * RPT, RPT)], wbuf)
    pltpu.sync_copy(wbuf, a_hbm.at[cid, pl.ds(sid * RPT, RPT)])


_fin_call = pl.kernel(
    _fin_body,
    out_type=jax.ShapeDtypeStruct((2, NPAD), f32),
    mesh=_mesh,
    compiler_params=pltpu.CompilerParams(use_tc_tiling_on_sc=False, needs_layout_passes=False),
    scratch_types=[
        pltpu.VMEM_SHARED((NPAD,), f32),
        pltpu.VMEM((N,), f32),
        pltpu.VMEM((K,), f32),
        pltpu.VMEM((K,), jnp.int32),
        pltpu.VMEM((K,), jnp.int32),
        pltpu.VMEM((RPT,), f32),
    ],
)


def _tc1_body(x_ref, w_ref, d0_ref, d1_ref, g_ref, dinv_ref):
    dinv = lax.rsqrt(d0_ref[0] + d1_ref[0] + 1.0)
    g_ref[...] = jnp.dot(x_ref[...], w_ref[...],
                         preferred_element_type=f32) * dinv
    dinv_ref[...] = dinv


def _tc2_body(p0_ref, p1_ref, g_ref, dinv_ref, b_ref, w_ref, o_ref):
    dinv = dinv_ref[...]
    h = jnp.maximum(
        dinv * (p0_ref[0] + p1_ref[0] + g_ref[...]) + b_ref[...], 0.0)
    o_ref[...] = jnp.dot(h, w_ref[...], preferred_element_type=f32) * dinv


def _tc4_body(a0_ref, a1_ref, gf_ref, dinv_ref, bf_ref, out_ref):
    out_ref[...] = dinv_ref[...] * (a0_ref[0] + a1_ref[0] + gf_ref[...]) \
        + bf_ref[...]


def _row_spec(w):
    return pl.BlockSpec((R, w), lambda i: (i, 0))


def _const_spec(h, w):
    return pl.BlockSpec((h, w), lambda i: (0, 0))


def _half_spec(c, w):
    return pl.BlockSpec((1, R, w), lambda i, c=c: (c, i, 0))


_GRID = N // R

_tc1_call = pl.pallas_call(
    _tc1_body,
    grid=(_GRID,),
    in_specs=[_row_spec(C), _const_spec(C, C), _half_spec(0, 1),
              _half_spec(1, 1)],
    out_specs=[_row_spec(C), _row_spec(1)],
    out_shape=[jax.ShapeDtypeStruct((N, C), f32),
               jax.ShapeDtypeStruct((N, 1), f32)],
)


def _make_tc2(cout):
    return pl.pallas_call(
        _tc2_body,
        grid=(_GRID,),
        in_specs=[_half_spec(0, C), _half_spec(1, C), _row_spec(C),
                  _row_spec(1), _const_spec(1, C), _const_spec(C, cout)],
        out_specs=_row_spec(cout),
        out_shape=jax.ShapeDtypeStruct((N, cout), f32),
    )


_tc2_call = _make_tc2(C)
_tc3_call = _make_tc2(1)

_tc4_call = pl.pallas_call(
    _tc4_body,
    grid=(_GRID,),
    in_specs=[_half_spec(0, 1), _half_spec(1, 1), _row_spec(1), _row_spec(1),
              _const_spec(1, 1)],
    out_specs=_row_spec(1),
    out_shape=jax.ShapeDtypeStruct((N, 1), f32),
)


@jax.jit
def kernel(x, edge_index, batch, W0, b0, W1, b1, Wf, bf):
    src = edge_index[0]
    dst = edge_index[1]

    d = _deg_call(dst).reshape(2, NPAD, 1)
    g0, dinv = _tc1_call(x, W0, d, d)
    p = _edge_call(g0, src, dst)
    g1 = _tc2_call(p, p, g0, dinv, b0.reshape(1, C), W1)
    q = _edge_call(g1, src, dst)
    gf = _tc3_call(q, q, g1, dinv, b1.reshape(1, C), Wf)
    a = _fin_call(gf.reshape(N), src, dst).reshape(2, NPAD, 1)
    out = _tc4_call(a, a, gf, dinv, bf.reshape(1, 1))
    return out


# speedup vs baseline: 19.5251x; 1.0218x over previous
import jax
import jax.numpy as jnp
from jax import lax
from jax.experimental import pallas as pl
from jax.experimental.pallas import tpu as pltpu
from jax.experimental.pallas import tpu_sc as plsc

N = 10000
E = 320000
C = 128
NPAD = 10240
RPT = 640
K = 80
R = 1000
EPW = E // 32

_mesh = plsc.VectorSubcoreMesh(core_axis_name="c", subcore_axis_name="s")
f32 = jnp.float32


def _fill_vec(ref, n, val):
    def body(j, _):
        ref[pl.ds(j * 16, 16)] = jnp.full((16,), val, f32)
        return 0
    lax.fori_loop(0, n // 16, body, 0)


NR = NPAD // C


def _zero_2d(ref, rows):
    def body(r, _):
        for c4 in range(C // 16):
            ref[r, pl.ds(c4 * 16, 16)] = jnp.zeros((16,), f32)
        return 0
    lax.fori_loop(0, rows, body, 0)


def _fill_iota(ref, n):
    def body(j, _):
        ref[pl.ds(j * 16, 16)] = jnp.arange(16, dtype=jnp.int32) + j * 16
        return 0
    lax.fori_loop(0, n // 16, body, 0)


def _drain_and_writeback(acc_l, acc_s, idt, out_hbm, cid, sid, wbuf):
    pltpu.sync_copy(acc_l, acc_s.at[idt], add=True)
    plsc.subcore_barrier()
    rows = NR // 16
    pltpu.sync_copy(acc_s.at[pl.ds(sid * rows, rows)], wbuf)
    pltpu.sync_copy(wbuf, out_hbm.at[cid, pl.ds(sid * rows, rows)])


def _deg_body(dst_hbm, d_hbm, acc_s, acc_l, idt, dst_v, wbuf):
    cid = lax.axis_index("c")
    sid = lax.axis_index("s")
    _zero_2d(acc_l, NR)
    _fill_iota(idt, NR)
    rows = NR // 16
    pltpu.sync_copy(acc_l.at[pl.ds(0, rows)], acc_s.at[pl.ds(sid * rows, rows)])
    plsc.subcore_barrier()

    wid = sid * 2 + cid
    ones16 = jnp.ones((16,), f32)

    def ebody(i, _):
        pltpu.sync_copy(dst_hbm.at[pl.ds(wid * EPW + i * K, K)], dst_v)
        for j in range(K // 16):
            d16 = dst_v[pl.ds(j * 16, 16)]
            row = lax.shift_right_logical(d16, 7)
            col = jnp.bitwise_and(d16, 127)
            plsc.addupdate_scatter(acc_l, [row, col], ones16)
        return 0
    lax.fori_loop(0, EPW // K, ebody, 0)
    plsc.subcore_barrier()
    _drain_and_writeback(acc_l, acc_s, idt, d_hbm, cid, sid, wbuf)


_deg_call = pl.kernel(
    _deg_body,
    out_type=jax.ShapeDtypeStruct((2, NR, C), f32),
    mesh=_mesh,
    compiler_params=pltpu.CompilerParams(use_tc_tiling_on_sc=False, needs_layout_passes=False),
    scratch_types=[
        pltpu.VMEM_SHARED((NR, C), f32),
        pltpu.VMEM((NR, C), f32),
        pltpu.VMEM((NR,), jnp.int32),
        pltpu.VMEM((K,), jnp.int32),
        pltpu.VMEM((NR // 16, C), f32),
    ],
)


def _edge_body(g_hbm, src_hbm, dst_hbm, p_hbm, acc_s,
               buf0, buf1, src_v0, src_v1, dst_v0, dst_v1, sem0, sem1):
    cid = lax.axis_index("c")
    sid = lax.axis_index("s")

    def zrow(r, _):
        for c4 in range(C // 16):
            buf0[r, pl.ds(c4 * 16, 16)] = jnp.zeros((16,), f32)
        return 0
    lax.fori_loop(0, K, zrow, 0)

    def zcp(k, _):
        pltpu.sync_copy(buf0, acc_s.at[pl.ds(sid * RPT + k * K, K)])
        return 0
    lax.fori_loop(0, RPT // K, zcp, 0)
    plsc.subcore_barrier()

    wid = sid * 2 + cid
    ebase = wid * EPW

    def fetch(block, sv, dv, bf, sem):
        pltpu.sync_copy(src_hbm.at[pl.ds(ebase + block * K, K)], sv)
        pltpu.sync_copy(dst_hbm.at[pl.ds(ebase + block * K, K)], dv)
        pltpu.async_copy(g_hbm.at[sv], bf, sem)

    def drain(sv, dv, bf, sem):
        pltpu.make_async_copy(g_hbm.at[sv], bf, sem).wait()
        pltpu.sync_copy(bf, acc_s.at[dv], add=True)

    fetch(0, src_v0, dst_v0, buf0, sem0)

    def pair(o, _):
        fetch(2 * o + 1, src_v1, dst_v1, buf1, sem1)
        drain(src_v0, dst_v0, buf0, sem0)
        fetch(2 * o + 2, src_v0, dst_v0, buf0, sem0)
        drain(src_v1, dst_v1, buf1, sem1)
        return 0
    lax.fori_loop(0, (EPW // K) // 2, pair, 0)
    drain(src_v0, dst_v0, buf0, sem0)
    plsc.subcore_barrier()

    def wb(k, _):
        base = sid * RPT + k * K
        pltpu.sync_copy(acc_s.at[pl.ds(base, K)], buf0)
        pltpu.sync_copy(buf0, p_hbm.at[cid, pl.ds(base, K)])
        return 0
    lax.fori_loop(0, RPT // K, wb, 0)


_edge_call = pl.kernel(
    _edge_body,
    out_type=jax.ShapeDtypeStruct((2, NPAD, C), f32),
    mesh=_mesh,
    compiler_params=pltpu.CompilerParams(use_tc_tiling_on_sc=False, needs_layout_passes=False),
    scratch_types=[
        pltpu.VMEM_SHARED((NPAD, C), f32),
        pltpu.VMEM((K, C), f32),
        pltpu.VMEM((K, C), f32),
        pltpu.VMEM((K,), jnp.int32),
        pltpu.VMEM((K,), jnp.int32),
        pltpu.VMEM((K,), jnp.int32),
        pltpu.VMEM((K,), jnp.int32),
        pltpu.SemaphoreType.DMA,
        pltpu.SemaphoreType.DMA,
    ],
)


def _fin_body(gf_hbm, src_hbm, dst_hbm, a_hbm,
              acc_s, acc_l, gf_v, idt, src_v, dst_v, wbuf):
    cid = lax.axis_index("c")
    sid = lax.axis_index("s")
    _zero_2d(acc_l, NR)
    _fill_iota(idt, NR)
    rows = NR // 16
    pltpu.sync_copy(acc_l.at[pl.ds(0, rows)], acc_s.at[pl.ds(sid * rows, rows)])
    pltpu.sync_copy(gf_hbm, gf_v)
    plsc.subcore_barrier()

    wid = sid * 2 + cid

    def ebody(i, _):
        base = wid * EPW + i * K
        pltpu.sync_copy(src_hbm.at[pl.ds(base, K)], src_v)
        pltpu.sync_copy(dst_hbm.at[pl.ds(base, K)], dst_v)
        for j in range(K // 16):
            s16 = src_v[pl.ds(j * 16, 16)]
            d16 = dst_v[pl.ds(j * 16, 16)]
            vals = plsc.load_gather(gf_v, [s16])
            row = lax.shift_right_logical(d16, 7)
            col = jnp.bitwise_and(d16, 127)
            plsc.addupdate_scatter(acc_l, [row, col], vals)
        return 0
    lax.fori_loop(0, EPW // K, ebody, 0)
    plsc.subcore_barrier()
    _drain_and_writeback(acc_l, acc_s, idt, a_hbm, cid, sid, wbuf)


_fin_call = pl.kernel(
    _fin_body,
    out_type=jax.ShapeDtypeStruct((2, NR, C), f32),
    mesh=_mesh,
    compiler_params=pltpu.CompilerParams(use_tc_tiling_on_sc=False, needs_layout_passes=False),
    scratch_types=[
        pltpu.VMEM_SHARED((NR, C), f32),
        pltpu.VMEM((NR, C), f32),
        pltpu.VMEM((N,), f32),
        pltpu.VMEM((NR,), jnp.int32),
        pltpu.VMEM((K,), jnp.int32),
        pltpu.VMEM((K,), jnp.int32),
        pltpu.VMEM((NR // 16, C), f32),
    ],
)


def _tc1_body(x_ref, w_ref, d0_ref, d1_ref, g_ref, dinv_ref):
    dinv = lax.rsqrt(d0_ref[0] + d1_ref[0] + 1.0)
    g_ref[...] = jnp.dot(x_ref[...], w_ref[...],
                         preferred_element_type=f32) * dinv
    dinv_ref[...] = dinv


def _tc2_body(p0_ref, p1_ref, g_ref, dinv_ref, b_ref, w_ref, o_ref):
    dinv = dinv_ref[...]
    h = jnp.maximum(
        dinv * (p0_ref[0] + p1_ref[0] + g_ref[...]) + b_ref[...], 0.0)
    o_ref[...] = jnp.dot(h, w_ref[...], preferred_element_type=f32) * dinv


def _tc4_body(a0_ref, a1_ref, gf_ref, dinv_ref, bf_ref, out_ref):
    out_ref[...] = dinv_ref[...] * (a0_ref[0] + a1_ref[0] + gf_ref[...]) \
        + bf_ref[...]


def _row_spec(w):
    return pl.BlockSpec((R, w), lambda i: (i, 0))


def _const_spec(h, w):
    return pl.BlockSpec((h, w), lambda i: (0, 0))


def _half_spec(c, w):
    return pl.BlockSpec((1, R, w), lambda i, c=c: (c, i, 0))


_GRID = N // R

_tc1_call = pl.pallas_call(
    _tc1_body,
    grid=(_GRID,),
    in_specs=[_row_spec(C), _const_spec(C, C), _half_spec(0, 1),
              _half_spec(1, 1)],
    out_specs=[_row_spec(C), _row_spec(1)],
    out_shape=[jax.ShapeDtypeStruct((N, C), f32),
               jax.ShapeDtypeStruct((N, 1), f32)],
)


def _make_tc2(cout):
    return pl.pallas_call(
        _tc2_body,
        grid=(_GRID,),
        in_specs=[_half_spec(0, C), _half_spec(1, C), _row_spec(C),
                  _row_spec(1), _const_spec(1, C), _const_spec(C, cout)],
        out_specs=_row_spec(cout),
        out_shape=jax.ShapeDtypeStruct((N, cout), f32),
    )


_tc2_call = _make_tc2(C)
_tc3_call = _make_tc2(1)

_tc4_call = pl.pallas_call(
    _tc4_body,
    grid=(_GRID,),
    in_specs=[_half_spec(0, 1), _half_spec(1, 1), _row_spec(1), _row_spec(1),
              _const_spec(1, 1)],
    out_specs=_row_spec(1),
    out_shape=jax.ShapeDtypeStruct((N, 1), f32),
)


@jax.jit
def kernel(x, edge_index, batch, W0, b0, W1, b1, Wf, bf):
    src = edge_index[0]
    dst = edge_index[1]

    d = _deg_call(dst).reshape(2, NPAD, 1)
    g0, dinv = _tc1_call(x, W0, d, d)
    p = _edge_call(g0, src, dst)
    g1 = _tc2_call(p, p, g0, dinv, b0.reshape(1, C), W1)
    q = _edge_call(g1, src, dst)
    gf = _tc3_call(q, q, g1, dinv, b1.reshape(1, C), Wf)
    a = _fin_call(gf.reshape(N), src, dst).reshape(2, NPAD, 1)
    out = _tc4_call(a, a, gf, dinv, bf.reshape(1, 1))
    return out


# speedup vs baseline: 32.9324x; 1.6867x over previous
import jax
import jax.numpy as jnp
from jax import lax
from jax.experimental import pallas as pl
from jax.experimental.pallas import tpu as pltpu
from jax.experimental.pallas import tpu_sc as plsc

N = 10000
E = 320000
C = 128
NPAD = 10240
RPT = 640
K = 80
R = 1000
EPW = E // 32
NBLK = EPW // 80

_mesh = plsc.VectorSubcoreMesh(core_axis_name="c", subcore_axis_name="s")
f32 = jnp.float32


def _fill_vec(ref, n, val):
    def body(j, _):
        ref[pl.ds(j * 16, 16)] = jnp.full((16,), val, f32)
        return 0
    lax.fori_loop(0, n // 16, body, 0)


NR = NPAD // C


def _zero_2d(ref, rows):
    def body(r, _):
        for c4 in range(C // 16):
            ref[r, pl.ds(c4 * 16, 16)] = jnp.zeros((16,), f32)
        return 0
    lax.fori_loop(0, rows, body, 0)


def _fill_iota(ref, n):
    def body(j, _):
        ref[pl.ds(j * 16, 16)] = jnp.arange(16, dtype=jnp.int32) + j * 16
        return 0
    lax.fori_loop(0, n // 16, body, 0)


def _drain_and_writeback(acc_l, acc_s, idt, out_hbm, cid, sid, wbuf):
    pltpu.sync_copy(acc_l, acc_s.at[idt], add=True)
    plsc.subcore_barrier()
    rows = NR // 16
    pltpu.sync_copy(acc_s.at[pl.ds(sid * rows, rows)], wbuf)
    pltpu.sync_copy(wbuf, out_hbm.at[cid, pl.ds(sid * rows, rows)])


def _deg_body(dst_hbm, d_hbm, acc_s, acc_l, idt, dbig, wbuf):
    cid = lax.axis_index("c")
    sid = lax.axis_index("s")
    _zero_2d(acc_l, NR)
    _fill_iota(idt, NR)
    rows = NR // 16
    pltpu.sync_copy(acc_l.at[pl.ds(0, rows)], acc_s.at[pl.ds(sid * rows, rows)])
    wid = sid * 2 + cid
    pltpu.sync_copy(dst_hbm.at[pl.ds(wid * NBLK, NBLK)], dbig)
    plsc.subcore_barrier()

    ones16 = jnp.ones((16,), f32)

    def ebody(i, _):
        for j in range(K // 16):
            d16 = dbig[i, pl.ds(j * 16, 16)]
            row = lax.shift_right_logical(d16, 7)
            col = jnp.bitwise_and(d16, 127)
            plsc.addupdate_scatter(acc_l, [row, col], ones16)
        return 0
    lax.fori_loop(0, NBLK, ebody, 0)
    plsc.subcore_barrier()
    _drain_and_writeback(acc_l, acc_s, idt, d_hbm, cid, sid, wbuf)


_deg_call = pl.kernel(
    _deg_body,
    out_type=jax.ShapeDtypeStruct((2, NR, C), f32),
    mesh=_mesh,
    compiler_params=pltpu.CompilerParams(use_tc_tiling_on_sc=False, needs_layout_passes=False),
    scratch_types=[
        pltpu.VMEM_SHARED((NR, C), f32),
        pltpu.VMEM((NR, C), f32),
        pltpu.VMEM((NR,), jnp.int32),
        pltpu.VMEM((NBLK, K), jnp.int32),
        pltpu.VMEM((NR // 16, C), f32),
    ],
)


CH = 25


def _edge_body(g_hbm, src_hbm, dst_hbm, p_hbm, acc_s,
               buf0, buf1, sbig, dbig, sem0, sem1):
    cid = lax.axis_index("c")
    sid = lax.axis_index("s")

    def zrow(r, _):
        for c4 in range(C // 16):
            buf0[r, pl.ds(c4 * 16, 16)] = jnp.zeros((16,), f32)
        return 0
    lax.fori_loop(0, K, zrow, 0)

    def zcp(k, _):
        pltpu.sync_copy(buf0, acc_s.at[pl.ds(sid * RPT + k * K, K)])
        return 0
    lax.fori_loop(0, RPT // K, zcp, 0)
    plsc.subcore_barrier()

    wid = sid * 2 + cid

    def start(b, bf, sem):
        pltpu.async_copy(g_hbm.at[sbig.at[b]], bf, sem)

    def drain(b, bf, sem):
        pltpu.make_async_copy(g_hbm.at[sbig.at[b]], bf, sem).wait()
        pltpu.sync_copy(bf, acc_s.at[dbig.at[b]], add=True)

    def chunk(c, _):
        rowbase = wid * NBLK + c * CH
        pltpu.sync_copy(src_hbm.at[pl.ds(rowbase, CH)], sbig)
        pltpu.sync_copy(dst_hbm.at[pl.ds(rowbase, CH)], dbig)
        start(0, buf0, sem0)

        def pair(o, _):
            start(2 * o + 1, buf1, sem1)
            drain(2 * o, buf0, sem0)
            start(2 * o + 2, buf0, sem0)
            drain(2 * o + 1, buf1, sem1)
            return 0
        lax.fori_loop(0, CH // 2, pair, 0)
        drain(CH - 1, buf0, sem0)
        return 0
    lax.fori_loop(0, NBLK // CH, chunk, 0)
    plsc.subcore_barrier()

    def wb(k, _):
        base = sid * RPT + k * K
        pltpu.sync_copy(acc_s.at[pl.ds(base, K)], buf0)
        pltpu.sync_copy(buf0, p_hbm.at[cid, pl.ds(base, K)])
        return 0
    lax.fori_loop(0, RPT // K, wb, 0)


_edge_call = pl.kernel(
    _edge_body,
    out_type=jax.ShapeDtypeStruct((2, NPAD, C), f32),
    mesh=_mesh,
    compiler_params=pltpu.CompilerParams(use_tc_tiling_on_sc=False, needs_layout_passes=False),
    scratch_types=[
        pltpu.VMEM_SHARED((NPAD, C), f32),
        pltpu.VMEM((K, C), f32),
        pltpu.VMEM((K, C), f32),
        pltpu.VMEM((CH, K), jnp.int32),
        pltpu.VMEM((CH, K), jnp.int32),
        pltpu.SemaphoreType.DMA,
        pltpu.SemaphoreType.DMA,
    ],
)


def _fin_body(gf_hbm, src_hbm, dst_hbm, a_hbm,
              acc_s, acc_l, gf_v, idt, sbig, dbig, wbuf):
    cid = lax.axis_index("c")
    sid = lax.axis_index("s")
    _zero_2d(acc_l, NR)
    _fill_iota(idt, NR)
    rows = NR // 16
    pltpu.sync_copy(acc_l.at[pl.ds(0, rows)], acc_s.at[pl.ds(sid * rows, rows)])
    pltpu.sync_copy(gf_hbm, gf_v)
    wid = sid * 2 + cid
    pltpu.sync_copy(src_hbm.at[pl.ds(wid * NBLK, NBLK)], sbig)
    pltpu.sync_copy(dst_hbm.at[pl.ds(wid * NBLK, NBLK)], dbig)
    plsc.subcore_barrier()

    def ebody(i, _):
        for j in range(K // 16):
            s16 = sbig[i, pl.ds(j * 16, 16)]
            d16 = dbig[i, pl.ds(j * 16, 16)]
            vals = plsc.load_gather(gf_v, [s16])
            row = lax.shift_right_logical(d16, 7)
            col = jnp.bitwise_and(d16, 127)
            plsc.addupdate_scatter(acc_l, [row, col], vals)
        return 0
    lax.fori_loop(0, NBLK, ebody, 0)
    plsc.subcore_barrier()
    _drain_and_writeback(acc_l, acc_s, idt, a_hbm, cid, sid, wbuf)


_fin_call = pl.kernel(
    _fin_body,
    out_type=jax.ShapeDtypeStruct((2, NR, C), f32),
    mesh=_mesh,
    compiler_params=pltpu.CompilerParams(use_tc_tiling_on_sc=False, needs_layout_passes=False),
    scratch_types=[
        pltpu.VMEM_SHARED((NR, C), f32),
        pltpu.VMEM((NR, C), f32),
        pltpu.VMEM((N,), f32),
        pltpu.VMEM((NR,), jnp.int32),
        pltpu.VMEM((NBLK, K), jnp.int32),
        pltpu.VMEM((NBLK, K), jnp.int32),
        pltpu.VMEM((NR // 16, C), f32),
    ],
)


def _tc1_body(x_ref, w_ref, d0_ref, d1_ref, g_ref, dinv_ref):
    dinv = lax.rsqrt(d0_ref[0] + d1_ref[0] + 1.0)
    g_ref[...] = jnp.dot(x_ref[...], w_ref[...],
                         preferred_element_type=f32) * dinv
    dinv_ref[...] = dinv


def _tc2_body(p0_ref, p1_ref, g_ref, dinv_ref, b_ref, w_ref, o_ref):
    dinv = dinv_ref[...]
    h = jnp.maximum(
        dinv * (p0_ref[0] + p1_ref[0] + g_ref[...]) + b_ref[...], 0.0)
    o_ref[...] = jnp.dot(h, w_ref[...], preferred_element_type=f32) * dinv


def _tc4_body(a0_ref, a1_ref, gf_ref, dinv_ref, bf_ref, out_ref):
    out_ref[...] = dinv_ref[...] * (a0_ref[0] + a1_ref[0] + gf_ref[...]) \
        + bf_ref[...]


def _row_spec(w):
    return pl.BlockSpec((R, w), lambda i: (i, 0))


def _const_spec(h, w):
    return pl.BlockSpec((h, w), lambda i: (0, 0))


def _half_spec(c, w):
    return pl.BlockSpec((1, R, w), lambda i, c=c: (c, i, 0))


_GRID = N // R

_tc1_call = pl.pallas_call(
    _tc1_body,
    grid=(_GRID,),
    in_specs=[_row_spec(C), _const_spec(C, C), _half_spec(0, 1),
              _half_spec(1, 1)],
    out_specs=[_row_spec(C), _row_spec(1)],
    out_shape=[jax.ShapeDtypeStruct((N, C), f32),
               jax.ShapeDtypeStruct((N, 1), f32)],
)


def _make_tc2(cout):
    return pl.pallas_call(
        _tc2_body,
        grid=(_GRID,),
        in_specs=[_half_spec(0, C), _half_spec(1, C), _row_spec(C),
                  _row_spec(1), _const_spec(1, C), _const_spec(C, cout)],
        out_specs=_row_spec(cout),
        out_shape=jax.ShapeDtypeStruct((N, cout), f32),
    )


_tc2_call = _make_tc2(C)
_tc3_call = _make_tc2(1)

_tc4_call = pl.pallas_call(
    _tc4_body,
    grid=(_GRID,),
    in_specs=[_half_spec(0, 1), _half_spec(1, 1), _row_spec(1), _row_spec(1),
              _const_spec(1, 1)],
    out_specs=_row_spec(1),
    out_shape=jax.ShapeDtypeStruct((N, 1), f32),
)


@jax.jit
def kernel(x, edge_index, batch, W0, b0, W1, b1, Wf, bf):
    src = edge_index[0].reshape(E // K, K)
    dst = edge_index[1].reshape(E // K, K)

    d = _deg_call(dst).reshape(2, NPAD, 1)
    g0, dinv = _tc1_call(x, W0, d, d)
    p = _edge_call(g0, src, dst)
    g1 = _tc2_call(p, p, g0, dinv, b0.reshape(1, C), W1)
    q = _edge_call(g1, src, dst)
    gf = _tc3_call(q, q, g1, dinv, b1.reshape(1, C), Wf)
    a = _fin_call(gf.reshape(N), src, dst).reshape(2, NPAD, 1)
    out = _tc4_call(a, a, gf, dinv, bf.reshape(1, 1))
    return out
